# Initial kernel scaffold; baseline (speedup 1.0000x reference)
import jax
import jax.numpy as jnp
from jax.experimental import pallas as pl


def kernel(x, edge_index, W1_rel, b1, W1_root, W2_rel, b2, W2_root):
    raise NotImplementedError("write your pallas kernel here")


# speedup vs baseline: 12.7630x; 12.7630x over previous
import functools

import jax
import jax.numpy as jnp
from jax import lax
from jax.experimental import pallas as pl
from jax.experimental.pallas import tpu as pltpu
from jax.experimental.pallas import tpu_sc as plsc

N_NODES = 10000
N_EDGES = 320000
D = 128

NC = 2
NS = 16
NW = NC * NS
EW = N_EDGES // NW
C = 128
K = 16
NBLK = 5
NCHUNK = NBLK * K
EWP = NCHUNK * C
PAD = EWP - EW
NDUMMY = 112
ACC_N = N_NODES + NDUMMY
ZROWS = ACC_N // NS
OUT_ROWS = 632

_MESH = plsc.VectorSubcoreMesh(core_axis_name="c", subcore_axis_name="s")


def _segsum_body(tab, srcs, dsts, out0, out1, acc,
                 srcb0, dstb0, srcb1, dstb1, buf0, buf1,
                 sem0, sem1, semi0, semi1):
    cid = lax.axis_index("c")
    sid = lax.axis_index("s")
    wid = sid * NC + cid

    srcbs = (srcb0, srcb1)
    dstbs = (dstb0, dstb1)
    bufs = (buf0, buf1)
    sems = (sem0, sem1)
    semis = (semi0, semi1)

    def _zero_row(r, carry):
        for j in range(D // 16):
            buf0[r, pl.ds(j * 16, 16)] = jnp.zeros((16,), jnp.float32)
        return carry

    lax.fori_loop(0, C, _zero_row, 0)
    z0 = sid * ZROWS
    for k in range(ZROWS // C):
        pltpu.sync_copy(buf0, acc.at[pl.ds(z0 + k * C, C)])
    rem = ZROWS % C
    if rem:
        pltpu.sync_copy(buf0.at[pl.ds(0, rem)],
                        acc.at[pl.ds(z0 + (ZROWS // C) * C, rem)])

    pltpu.sync_copy(srcs.at[wid, pl.ds(0, K)], srcb0)
    pltpu.sync_copy(dsts.at[wid, pl.ds(0, K)], dstb0)
    plsc.subcore_barrier()

    def _idx_start(b, side):
        pltpu.async_copy(srcs.at[wid, pl.ds(b * K, K)], srcbs[side], semis[side])
        pltpu.async_copy(dsts.at[wid, pl.ds(b * K, K)], dstbs[side], semis[side])

    def _idx_wait(b, side):
        pltpu.make_async_copy(srcs.at[wid, pl.ds(b * K, K)], srcbs[side],
                              semis[side]).wait()
        pltpu.make_async_copy(dsts.at[wid, pl.ds(b * K, K)], dstbs[side],
                              semis[side]).wait()

    pltpu.async_copy(tab.at[srcb0.at[0]], buf0, sem0)

    for b in range(NBLK):
        cs, cd = srcbs[b % 2], dstbs[b % 2]
        nside = (b + 1) % 2
        if b + 1 < NBLK:
            _idx_start(b + 1, nside)
        for j in range(K):
            g = b * K + j
            if g + 1 < NCHUNK:
                if j + 1 < K:
                    nidx = cs.at[j + 1]
                else:
                    _idx_wait(b + 1, nside)
                    nidx = srcbs[nside].at[0]
                pltpu.async_copy(tab.at[nidx], bufs[(g + 1) % 2],
                                 sems[(g + 1) % 2])
            pltpu.make_async_copy(tab.at[cs.at[j]], bufs[g % 2],
                                  sems[g % 2]).wait()
            pltpu.sync_copy(bufs[g % 2], acc.at[cd.at[j]], add=True)

    plsc.subcore_barrier()

    o0 = sid * OUT_ROWS
    last = NS * OUT_ROWS - OUT_ROWS
    tail = N_NODES - last

    @pl.when(jnp.logical_and(cid == 0, sid < NS - 1))
    def _():
        pltpu.sync_copy(acc.at[pl.ds(o0, OUT_ROWS)], out0.at[pl.ds(o0, OUT_ROWS)])

    @pl.when(jnp.logical_and(cid == 0, sid == NS - 1))
    def _():
        pltpu.sync_copy(acc.at[pl.ds(last, tail)], out0.at[pl.ds(last, tail)])

    @pl.when(jnp.logical_and(cid == 1, sid < NS - 1))
    def _():
        pltpu.sync_copy(acc.at[pl.ds(o0, OUT_ROWS)], out1.at[pl.ds(o0, OUT_ROWS)])

    @pl.when(jnp.logical_and(cid == 1, sid == NS - 1))
    def _():
        pltpu.sync_copy(acc.at[pl.ds(last, tail)], out1.at[pl.ds(last, tail)])


_segsum_sc = pl.kernel(
    _segsum_body,
    out_type=(
        jax.ShapeDtypeStruct((N_NODES, D), jnp.float32),
        jax.ShapeDtypeStruct((N_NODES, D), jnp.float32),
    ),
    mesh=_MESH,
    scratch_types=[
        pltpu.VMEM_SHARED((ACC_N, D), jnp.float32),
        pltpu.VMEM((K, C), jnp.int32),
        pltpu.VMEM((K, C), jnp.int32),
        pltpu.VMEM((K, C), jnp.int32),
        pltpu.VMEM((K, C), jnp.int32),
        pltpu.VMEM((C, D), jnp.float32),
        pltpu.VMEM((C, D), jnp.float32),
        pltpu.SemaphoreType.DMA,
        pltpu.SemaphoreType.DMA,
        pltpu.SemaphoreType.DMA,
        pltpu.SemaphoreType.DMA,
    ],
)


def _affine_body(p0, p1, xr, wr, wo, br, o, *, relu):
    dn = (((1,), (1,)), ((), ()))
    agg = p0[...] + p1[...]
    y = lax.dot_general(agg, wr[...], dn, preferred_element_type=jnp.float32)
    y = y + lax.dot_general(xr[...], wo[...], dn, preferred_element_type=jnp.float32)
    y = y + br[...]
    if relu:
        y = jnp.maximum(y, 0.0)
    o[...] = y


def _affine(p0, p1, x, w_rel, w_root, b, relu):
    bn = 1000
    grid = N_NODES // bn
    row = lambda i: (i, 0)
    zero = lambda i: (0, 0)
    return pl.pallas_call(
        functools.partial(_affine_body, relu=relu),
        grid=(grid,),
        in_specs=[
            pl.BlockSpec((bn, D), row),
            pl.BlockSpec((bn, D), row),
            pl.BlockSpec((bn, D), row),
            pl.BlockSpec((D, D), zero),
            pl.BlockSpec((D, D), zero),
            pl.BlockSpec((1, D), zero),
        ],
        out_specs=pl.BlockSpec((bn, D), row),
        out_shape=jax.ShapeDtypeStruct((N_NODES, D), jnp.float32),
    )(p0, p1, x, w_rel, w_root, b)


def kernel(x, edge_index, W1_rel, b1, W1_root, W2_rel, b2, W2_root):
    src = edge_index[0].astype(jnp.int32).reshape(NW, EW)
    dst = edge_index[1].astype(jnp.int32).reshape(NW, EW)
    pad_ar = jnp.arange(PAD, dtype=jnp.int32)
    pad_src = jnp.broadcast_to((pad_ar * 89) % N_NODES, (NW, PAD))
    pad_dst = jnp.broadcast_to(N_NODES + pad_ar % NDUMMY, (NW, PAD))
    src3 = jnp.concatenate([src, pad_src], axis=1).reshape(NW, NCHUNK, C)
    dst3 = jnp.concatenate([dst, pad_dst], axis=1).reshape(NW, NCHUNK, C)

    b1r = b1.reshape(1, D)
    b2r = b2.reshape(1, D)

    p0, p1 = _segsum_sc(x, src3, dst3)
    h = _affine(p0, p1, x, W1_rel, W1_root, b1r, relu=True)
    q0, q1 = _segsum_sc(h, src3, dst3)
    return _affine(q0, q1, h, W2_rel, W2_root, b2r, relu=False)


# speedup vs baseline: 12.8263x; 1.0050x over previous
import functools

import jax
import jax.numpy as jnp
from jax import lax
from jax.experimental import pallas as pl
from jax.experimental.pallas import tpu as pltpu
from jax.experimental.pallas import tpu_sc as plsc

N_NODES = 10000
N_EDGES = 320000
D = 128

NC = 2
NS = 16
NW = NC * NS
EW = N_EDGES // NW
C = 128
K = 16
NBLK = 5
NCHUNK = NBLK * K
EWP = NCHUNK * C
PAD = EWP - EW
NDUMMY = 112
ACC_N = N_NODES + NDUMMY
ZROWS = ACC_N // NS
OUT_ROWS = 632

_MESH = plsc.VectorSubcoreMesh(core_axis_name="c", subcore_axis_name="s")


def _segsum_body(tab, srcs, dsts, out0, out1, acc,
                 srcb0, dstb0, srcb1, dstb1, buf0, buf1,
                 sem0, sem1, semi0, semi1):
    cid = lax.axis_index("c")
    sid = lax.axis_index("s")
    wid = sid * NC + cid

    srcbs = (srcb0, srcb1)
    dstbs = (dstb0, dstb1)
    bufs = (buf0, buf1)
    sems = (sem0, sem1)
    semis = (semi0, semi1)

    pltpu.sync_copy(srcs.at[wid, pl.ds(0, K)], srcb0)
    pltpu.sync_copy(dsts.at[wid, pl.ds(0, K)], dstb0)
    pltpu.async_copy(tab.at[srcb0.at[0]], buf0, sem0)

    def _zero_row(r, carry):
        for j in range(D // 16):
            buf1[r, pl.ds(j * 16, 16)] = jnp.zeros((16,), jnp.float32)
        return carry

    lax.fori_loop(0, C, _zero_row, 0)
    z0 = sid * ZROWS
    for k in range(ZROWS // C):
        pltpu.sync_copy(buf1, acc.at[pl.ds(z0 + k * C, C)])
    rem = ZROWS % C
    if rem:
        pltpu.sync_copy(buf1.at[pl.ds(0, rem)],
                        acc.at[pl.ds(z0 + (ZROWS // C) * C, rem)])
    plsc.subcore_barrier()

    def _idx_start(b, side):
        pltpu.async_copy(srcs.at[wid, pl.ds(b * K, K)], srcbs[side], semis[side])
        pltpu.async_copy(dsts.at[wid, pl.ds(b * K, K)], dstbs[side], semis[side])

    def _idx_wait(b, side):
        pltpu.make_async_copy(srcs.at[wid, pl.ds(b * K, K)], srcbs[side],
                              semis[side]).wait()
        pltpu.make_async_copy(dsts.at[wid, pl.ds(b * K, K)], dstbs[side],
                              semis[side]).wait()

    for b in range(NBLK):
        cs, cd = srcbs[b % 2], dstbs[b % 2]
        nside = (b + 1) % 2
        if b + 1 < NBLK:
            _idx_start(b + 1, nside)
        for j in range(K):
            g = b * K + j
            if g + 1 < NCHUNK:
                if j + 1 < K:
                    nidx = cs.at[j + 1]
                else:
                    _idx_wait(b + 1, nside)
                    nidx = srcbs[nside].at[0]
                pltpu.async_copy(tab.at[nidx], bufs[(g + 1) % 2],
                                 sems[(g + 1) % 2])
            pltpu.make_async_copy(tab.at[cs.at[j]], bufs[g % 2],
                                  sems[g % 2]).wait()
            pltpu.sync_copy(bufs[g % 2], acc.at[cd.at[j]], add=True)

    plsc.subcore_barrier()

    o0 = sid * OUT_ROWS
    last = NS * OUT_ROWS - OUT_ROWS
    tail = N_NODES - last

    @pl.when(jnp.logical_and(cid == 0, sid < NS - 1))
    def _():
        pltpu.sync_copy(acc.at[pl.ds(o0, OUT_ROWS)], out0.at[pl.ds(o0, OUT_ROWS)])

    @pl.when(jnp.logical_and(cid == 0, sid == NS - 1))
    def _():
        pltpu.sync_copy(acc.at[pl.ds(last, tail)], out0.at[pl.ds(last, tail)])

    @pl.when(jnp.logical_and(cid == 1, sid < NS - 1))
    def _():
        pltpu.sync_copy(acc.at[pl.ds(o0, OUT_ROWS)], out1.at[pl.ds(o0, OUT_ROWS)])

    @pl.when(jnp.logical_and(cid == 1, sid == NS - 1))
    def _():
        pltpu.sync_copy(acc.at[pl.ds(last, tail)], out1.at[pl.ds(last, tail)])


_segsum_sc = pl.kernel(
    _segsum_body,
    out_type=(
        jax.ShapeDtypeStruct((N_NODES, D), jnp.float32),
        jax.ShapeDtypeStruct((N_NODES, D), jnp.float32),
    ),
    mesh=_MESH,
    scratch_types=[
        pltpu.VMEM_SHARED((ACC_N, D), jnp.float32),
        pltpu.VMEM((K, C), jnp.int32),
        pltpu.VMEM((K, C), jnp.int32),
        pltpu.VMEM((K, C), jnp.int32),
        pltpu.VMEM((K, C), jnp.int32),
        pltpu.VMEM((C, D), jnp.float32),
        pltpu.VMEM((C, D), jnp.float32),
        pltpu.SemaphoreType.DMA,
        pltpu.SemaphoreType.DMA,
        pltpu.SemaphoreType.DMA,
        pltpu.SemaphoreType.DMA,
    ],
)


_BN = 1000
_ROW = lambda i: (i, 0)
_ZERO = lambda i: (0, 0)


def _root_body(xr, wo, br, o):
    dn = (((1,), (1,)), ((), ()))
    o[...] = lax.dot_general(xr[...], wo[...], dn,
                             preferred_element_type=jnp.float32) + br[...]


def _root_affine(x, w_root, b):
    return pl.pallas_call(
        _root_body,
        grid=(N_NODES // _BN,),
        in_specs=[
            pl.BlockSpec((_BN, D), _ROW),
            pl.BlockSpec((D, D), _ZERO),
            pl.BlockSpec((1, D), _ZERO),
        ],
        out_specs=pl.BlockSpec((_BN, D), _ROW),
        out_shape=jax.ShapeDtypeStruct((N_NODES, D), jnp.float32),
    )(x, w_root, b)


def _rel_body(p0, p1, rr, wr, o, *, relu):
    dn = (((1,), (1,)), ((), ()))
    agg = p0[...] + p1[...]
    y = lax.dot_general(agg, wr[...], dn,
                        preferred_element_type=jnp.float32) + rr[...]
    if relu:
        y = jnp.maximum(y, 0.0)
    o[...] = y


def _rel_affine(p0, p1, r, w_rel, relu):
    return pl.pallas_call(
        functools.partial(_rel_body, relu=relu),
        grid=(N_NODES // _BN,),
        in_specs=[
            pl.BlockSpec((_BN, D), _ROW),
            pl.BlockSpec((_BN, D), _ROW),
            pl.BlockSpec((_BN, D), _ROW),
            pl.BlockSpec((D, D), _ZERO),
        ],
        out_specs=pl.BlockSpec((_BN, D), _ROW),
        out_shape=jax.ShapeDtypeStruct((N_NODES, D), jnp.float32),
    )(p0, p1, r, w_rel)


def kernel(x, edge_index, W1_rel, b1, W1_root, W2_rel, b2, W2_root):
    src = edge_index[0].astype(jnp.int32).reshape(NW, EW)
    dst = edge_index[1].astype(jnp.int32).reshape(NW, EW)
    pad_ar = jnp.arange(PAD, dtype=jnp.int32)
    pad_src = jnp.broadcast_to((pad_ar * 89) % N_NODES, (NW, PAD))
    pad_dst = jnp.broadcast_to(N_NODES + pad_ar % NDUMMY, (NW, PAD))
    src3 = jnp.concatenate([src, pad_src], axis=1).reshape(NW, NCHUNK, C)
    dst3 = jnp.concatenate([dst, pad_dst], axis=1).reshape(NW, NCHUNK, C)

    r1 = _root_affine(x, W1_root, b1.reshape(1, D))
    p0, p1 = _segsum_sc(x, src3, dst3)
    h = _rel_affine(p0, p1, r1, W1_rel, relu=True)

    r2 = _root_affine(h, W2_root, b2.reshape(1, D))
    q0, q1 = _segsum_sc(h, src3, dst3)
    return _rel_affine(q0, q1, r2, W2_rel, relu=False)
